# baseline (speedup 1.0000x reference)
import numpy as np
import jax
import jax.numpy as jnp
from jax import lax
from jax.experimental import pallas as pl
from jax.experimental.pallas import tpu as pltpu
from jax.experimental.pallas import tpu_sc as plsc

F = 26
V = 1000
D = 128
B = 4096

NW = 32
ROWS = B * F
RPW = ROWS // NW
CH = 208
RECS = CH // F
NCH = RPW // CH
NBUF = 4
DIST = 2

_FOFF = np.asarray((np.arange(CH) % F) * V, dtype=np.int32)


def _fuse_body(tab_ref, bias_ref, out_ref):
    out_ref[...] = tab_ref[...] + bias_ref[...]


def _fuse(tables, bias):
    return pl.pallas_call(
        _fuse_body,
        grid=(F // 2,),
        in_specs=[
            pl.BlockSpec((2, V, D), lambda f: (f, 0, 0)),
            pl.BlockSpec((2, 1, D), lambda f: (f, 0, 0)),
        ],
        out_specs=pl.BlockSpec((2, V, D), lambda f: (f, 0, 0)),
        out_shape=jax.ShapeDtypeStruct((F, V, D), jnp.float32),
    )(tables, bias.reshape(F, 1, D))


def _gather_body(x_hbm, foff_hbm, tab_hbm, out_hbm,
                 gb0, gb1, gb2, gb3,
                 xall, iball, foff_v,
                 gs0, gs1, gs2, gs3,
                 ss0, ss1, ss2, ss3):
    wid = lax.axis_index("s") * 2 + lax.axis_index("c")
    base = wid * RPW
    rbase0 = wid * (RPW // F)

    GB = (gb0, gb1, gb2, gb3)
    GS = (gs0, gs1, gs2, gs3)
    SS = (ss0, ss1, ss2, ss3)

    pltpu.sync_copy(foff_hbm, foff_v)
    pltpu.sync_copy(x_hbm.at[pl.ds(base, RPW)], xall)
    for c in range(NCH):
        for i in range(CH // 16):
            sl = pl.ds(c * CH + i * 16, 16)
            iball[sl] = xall[sl] + foff_v[pl.ds(i * 16, 16)]

    def wait_store(q):
        pltpu.make_async_copy(tab_hbm.at[pl.ds(0, CH)], GB[q], SS[q]).wait()

    def fetch(c, q, wait):
        if wait:
            wait_store(q)
        pltpu.async_copy(tab_hbm.at[iball.at[pl.ds(c * CH, CH)]], GB[q], GS[q])

    def body(c, p):
        pltpu.make_async_copy(tab_hbm.at[iball.at[pl.ds(c * CH, CH)]],
                              GB[p], GS[p]).wait()
        rb = rbase0 + c * RECS
        for r in range(RECS):
            pltpu.async_copy(GB[p].at[pl.ds(r * F, F)], out_hbm.at[rb + r], SS[p])

    for c in range(DIST):
        fetch(c, c % NBUF, wait=False)

    for c in range(NCH):
        body(c, c % NBUF)
        if c + DIST < NCH:
            fetch(c + DIST, (c + DIST) % NBUF, wait=(c + DIST >= NBUF))

    for q in range(NBUF):
        wait_store(q)


def kernel(x, tables, bias):
    x_flat = x.reshape(ROWS).astype(jnp.int32)
    fused = _fuse(tables, bias).reshape(F * V, D)
    foff = jnp.asarray(_FOFF)

    mesh = plsc.VectorSubcoreMesh(core_axis_name="c", subcore_axis_name="s")
    run = pl.kernel(
        _gather_body,
        out_type=jax.ShapeDtypeStruct((B, F, D), jnp.float32),
        mesh=mesh,
        scratch_types=(
            [pltpu.VMEM((CH, D), jnp.float32) for _ in range(NBUF)]
            + [pltpu.VMEM((RPW,), jnp.int32),
               pltpu.VMEM((RPW,), jnp.int32),
               pltpu.VMEM((CH,), jnp.int32)]
            + [pltpu.SemaphoreType.DMA for _ in range(NBUF)]
            + [pltpu.SemaphoreType.DMA for _ in range(NBUF)]
        ),
    )
    return run(x_flat, foff, fused)


# speedup vs baseline: 1.0057x; 1.0057x over previous
import numpy as np
import jax
import jax.numpy as jnp
from jax import lax
from jax.experimental import pallas as pl
from jax.experimental.pallas import tpu as pltpu
from jax.experimental.pallas import tpu_sc as plsc

F = 26
V = 1000
D = 128
B = 4096

NW = 32
ROWS = B * F
RPW = ROWS // NW
CH = 208
RECS = CH // F
NCH = RPW // CH
NBUF = 4
DIST = 2

_FOFF = np.asarray((np.arange(CH) % F) * V, dtype=np.int32)


def _fuse_body(tab_ref, bias_ref, out_ref):
    out_ref[...] = tab_ref[...] + bias_ref[...]


def _fuse(tables, bias):
    return pl.pallas_call(
        _fuse_body,
        grid=(F // 2,),
        in_specs=[
            pl.BlockSpec((2, V, D), lambda f: (f, 0, 0)),
            pl.BlockSpec((2, 1, D), lambda f: (f, 0, 0)),
        ],
        out_specs=pl.BlockSpec((2, V, D), lambda f: (f, 0, 0)),
        out_shape=jax.ShapeDtypeStruct((F, V, D), jnp.float32),
    )(tables, bias.reshape(F, 1, D))


def _gather_body(x_hbm, foff_hbm, tab_hbm, out_hbm,
                 xb0, xb1, xb2, xb3,
                 gb0, gb1, gb2, gb3,
                 foff_v,
                 gs0, gs1, gs2, gs3,
                 ss0, ss1, ss2, ss3):
    wid = lax.axis_index("s") * 2 + lax.axis_index("c")
    base = wid * RPW
    rbase0 = wid * (RPW // F)

    XB = (xb0, xb1, xb2, xb3)
    GB = (gb0, gb1, gb2, gb3)
    GS = (gs0, gs1, gs2, gs3)
    SS = (ss0, ss1, ss2, ss3)

    pltpu.sync_copy(foff_hbm, foff_v)

    def wait_store(q):
        pltpu.make_async_copy(tab_hbm.at[pl.ds(0, CH)], GB[q], SS[q]).wait()

    def fetch(c, q, wait):
        if wait:
            wait_store(q)
        pltpu.sync_copy(x_hbm.at[pl.ds(base + c * CH, CH)], XB[q])
        for i in range(CH // 16):
            sl = pl.ds(i * 16, 16)
            XB[q][sl] = XB[q][sl] + foff_v[sl]
        pltpu.async_copy(tab_hbm.at[XB[q]], GB[q], GS[q])

    def body(c, p):
        pltpu.make_async_copy(tab_hbm.at[XB[p]], GB[p], GS[p]).wait()
        rb = rbase0 + c * RECS
        for r in range(RECS):
            pltpu.async_copy(GB[p].at[pl.ds(r * F, F)], out_hbm.at[rb + r], SS[p])

    for c in range(DIST):
        fetch(c, c % NBUF, wait=False)

    for c in range(NCH):
        body(c, c % NBUF)
        if c + DIST < NCH:
            fetch(c + DIST, (c + DIST) % NBUF, wait=(c + DIST >= NBUF))

    for q in range(NBUF):
        wait_store(q)


def kernel(x, tables, bias):
    x_flat = x.reshape(ROWS).astype(jnp.int32)
    fused = _fuse(tables, bias).reshape(F * V, D)
    foff = jnp.asarray(_FOFF)

    mesh = plsc.VectorSubcoreMesh(core_axis_name="c", subcore_axis_name="s")
    run = pl.kernel(
        _gather_body,
        out_type=jax.ShapeDtypeStruct((B, F, D), jnp.float32),
        mesh=mesh,
        scratch_types=(
            [pltpu.VMEM((CH,), jnp.int32) for _ in range(NBUF)]
            + [pltpu.VMEM((CH, D), jnp.float32) for _ in range(NBUF)]
            + [pltpu.VMEM((CH,), jnp.int32)]
            + [pltpu.SemaphoreType.DMA for _ in range(NBUF)]
            + [pltpu.SemaphoreType.DMA for _ in range(NBUF)]
        ),
    )
    return run(x_flat, foff, fused)
